# Initial kernel scaffold; baseline (speedup 1.0000x reference)
import jax
import jax.numpy as jnp
from jax.experimental import pallas as pl


def kernel(edge_index, edge_attr, node_emb, W1, b1, W2, b2):
    raise NotImplementedError("write your pallas kernel here")


# speedup vs baseline: 12.7771x; 12.7771x over previous
import functools

import jax
import jax.numpy as jnp
from jax import lax
from jax.experimental import pallas as pl
from jax.experimental.pallas import tpu as pltpu
from jax.experimental.pallas import tpu_sc as plsc

N = 10000
D = 128
E = 320000
NC = 2
NS = 16
NW = NC * NS
LANES = 128
K = -(-E // (NW * LANES))
EPAD = NW * K * LANES
NPAD = 10240
RPT = NPAD // NS
BR = 1000
GRID = N // BR

_mesh = plsc.VectorSubcoreMesh(core_axis_name="c", subcore_axis_name="s",
                               num_cores=NC, num_subcores=NS)


@functools.partial(
    pl.kernel,
    out_type=jax.ShapeDtypeStruct((NC, NPAD, D), jnp.float32),
    mesh=_mesh,
    scratch_types=[
        pltpu.VMEM((K, LANES), jnp.int32),
        pltpu.VMEM((LANES, D), jnp.float32),
        pltpu.VMEM_SHARED((NPAD, D), jnp.float32),
    ],
)
def _sc_degree(dst_hbm, ones_hbm, zero_hbm, out_hbm, idx_v, ones_v, acc):
    c = lax.axis_index("c")
    s = lax.axis_index("s")
    wid = s * NC + c
    pltpu.sync_copy(dst_hbm.at[wid], idx_v)
    pltpu.sync_copy(ones_hbm, ones_v)
    pltpu.sync_copy(zero_hbm, acc.at[pl.ds(s * RPT, RPT)])
    plsc.subcore_barrier()

    def step(j, carry):
        pltpu.sync_copy(ones_v, acc.at[idx_v.at[j]], add=True)
        return carry

    lax.fori_loop(0, K, step, 0)
    plsc.subcore_barrier()
    pltpu.sync_copy(acc.at[pl.ds(s * RPT, RPT)], out_hbm.at[c, pl.ds(s * RPT, RPT)])


@functools.partial(
    pl.kernel,
    out_type=jax.ShapeDtypeStruct((NC, NPAD, D), jnp.float32),
    mesh=_mesh,
    scratch_types=[
        pltpu.VMEM((K, LANES), jnp.int32),
        pltpu.VMEM((K, LANES), jnp.int32),
        pltpu.VMEM((LANES, D), jnp.float32),
        pltpu.VMEM_SHARED((NPAD, D), jnp.float32),
        pltpu.SemaphoreType.DMA,
    ],
)
def _sc_aggregate(z_hbm, src_hbm, dst_hbm, zero_hbm, out_hbm,
                  src_v, dst_v, rows_v, acc, sem):
    c = lax.axis_index("c")
    s = lax.axis_index("s")
    wid = s * NC + c
    pltpu.sync_copy(src_hbm.at[wid], src_v)
    pltpu.sync_copy(dst_hbm.at[wid], dst_v)
    pltpu.sync_copy(zero_hbm, acc.at[pl.ds(s * RPT, RPT)])
    plsc.subcore_barrier()

    def step(j, carry):
        pltpu.async_copy(z_hbm.at[src_v.at[j]], rows_v, sem).wait()
        pltpu.sync_copy(rows_v, acc.at[dst_v.at[j]], add=True)
        return carry

    lax.fori_loop(0, K, step, 0)
    plsc.subcore_barrier()
    pltpu.sync_copy(acc.at[pl.ds(s * RPT, RPT)], out_hbm.at[c, pl.ds(s * RPT, RPT)])


def _tc1_body(deg0_ref, deg1_ref, x_ref, w_ref, z_ref, dinv_ref):
    deg = deg0_ref[...] + deg1_ref[...] + 1.0
    dinv = lax.rsqrt(deg)
    dinv_ref[...] = dinv
    z_ref[...] = jnp.dot(x_ref[...] * dinv, w_ref[...],
                         preferred_element_type=jnp.float32)


def _tc2_body(p0_ref, p1_ref, z_ref, dinv_ref, w_ref, b_ref, z2_ref):
    dinv = dinv_ref[...]
    h = jnp.maximum((p0_ref[...] + p1_ref[...] + z_ref[...]) * dinv + b_ref[...],
                    0.0)
    z2_ref[...] = jnp.dot(h * dinv, w_ref[...], preferred_element_type=jnp.float32)


def _tc3_body(p0_ref, p1_ref, z_ref, dinv_ref, b_ref, out_ref):
    out_ref[...] = ((p0_ref[...] + p1_ref[...] + z_ref[...]) * dinv_ref[...]
                    + b_ref[...])


_col = pl.BlockSpec((BR, 1), lambda i: (i, 0))
_row = pl.BlockSpec((BR, D), lambda i: (i, 0))
_mat = pl.BlockSpec((D, D), lambda i: (0, 0))
_vec = pl.BlockSpec((1, D), lambda i: (0, 0))


def _tc1(deg0, deg1, x, w):
    return pl.pallas_call(
        _tc1_body,
        grid=(GRID,),
        in_specs=[_col, _col, _row, _mat],
        out_specs=[_row, _col],
        out_shape=[jax.ShapeDtypeStruct((N, D), jnp.float32),
                   jax.ShapeDtypeStruct((N, 1), jnp.float32)],
    )(deg0, deg1, x, w)


def _tc2(p0, p1, z, dinv, w, b):
    return pl.pallas_call(
        _tc2_body,
        grid=(GRID,),
        in_specs=[_row, _row, _row, _col, _mat, _vec],
        out_specs=_row,
        out_shape=jax.ShapeDtypeStruct((N, D), jnp.float32),
    )(p0, p1, z, dinv, w, b)


def _tc3(p0, p1, z, dinv, b):
    return pl.pallas_call(
        _tc3_body,
        grid=(GRID,),
        in_specs=[_row, _row, _row, _col, _vec],
        out_specs=_row,
        out_shape=jax.ShapeDtypeStruct((N, D), jnp.float32),
    )(p0, p1, z, dinv, b)


def kernel(edge_index, edge_attr, node_emb, W1, b1, W2, b2):
    del edge_attr
    ei = edge_index.astype(jnp.int32)
    pad = EPAD - E
    src = jnp.concatenate([ei[0], jnp.zeros((pad,), jnp.int32)]).reshape(NW, K, LANES)
    dst = jnp.concatenate([ei[1], jnp.full((pad,), N, jnp.int32)]).reshape(NW, K, LANES)

    onesD = jnp.ones((LANES, D), jnp.float32)
    zeroD = jnp.zeros((RPT, D), jnp.float32)

    degp = _sc_degree(dst, onesD, zeroD)
    deg0 = degp[0, :N, 0:1]
    deg1 = degp[1, :N, 0:1]

    z1, dinv = _tc1(deg0, deg1, node_emb, W1)

    agg1 = _sc_aggregate(z1, src, dst, zeroD)
    z2 = _tc2(agg1[0, :N, :], agg1[1, :N, :], z1, dinv, W2,
              b1.reshape(1, D))

    agg2 = _sc_aggregate(z2, src, dst, zeroD)
    out = _tc3(agg2[0, :N, :], agg2[1, :N, :], z2, dinv, b2.reshape(1, D))
    return out
